# baseline (speedup 1.0000x reference)
import functools
import math

import jax
import jax.numpy as jnp
from jax import lax
from jax.experimental import pallas as pl
from jax.experimental.pallas import tpu as pltpu
from jax.experimental.pallas import tpu_sc as plsc

D_MODEL = 128
MAX_SEQ_LEN = 200
BATCH = 1024
_SCALE = math.sqrt(float(D_MODEL))

NC = 2
NS = 16
NW = NC * NS
LANES = 16
NVEC = D_MODEL // LANES
U_ROWS = 100
UNITS = BATCH * MAX_SEQ_LEN // (NW * U_ROWS)
NBUF = 4
AHEAD = NBUF - 1


def _pos_encoding():
    even_i = jnp.arange(0, D_MODEL, 2, dtype=jnp.float32)
    denominator = jnp.power(even_i, even_i / D_MODEL)
    position = jnp.arange(MAX_SEQ_LEN, dtype=jnp.float32).reshape(MAX_SEQ_LEN, 1)
    even_pe = jnp.sin(position / denominator)
    odd_pe = jnp.cos(position / denominator)
    stacked = jnp.stack([even_pe, odd_pe], axis=-1)
    return stacked.reshape(MAX_SEQ_LEN, D_MODEL)


def _embed_encode(x4, pe, table):
    mesh = plsc.VectorSubcoreMesh(core_axis_name="c", subcore_axis_name="s")

    scratch = [
        pltpu.VMEM((MAX_SEQ_LEN, D_MODEL), jnp.float32),
        pltpu.VMEM((UNITS, U_ROWS), jnp.int32),
    ]
    scratch += [pltpu.VMEM((U_ROWS, D_MODEL), jnp.float32)
                for _ in range(NBUF)]
    scratch += [pltpu.SemaphoreType.DMA for _ in range(2 * NBUF)]

    @functools.partial(
        pl.kernel,
        out_type=jax.ShapeDtypeStruct((NW, UNITS, U_ROWS, D_MODEL), jnp.float32),
        mesh=mesh,
        scratch_types=scratch,
    )
    def k(x_hbm, pe_hbm, table_hbm, out_hbm, pe_v, idx_all, *rest):
        rows = rest[:NBUF]
        gsems = rest[NBUF:2 * NBUF]
        osems = rest[2 * NBUF:3 * NBUF]

        wid = lax.axis_index("s") * NC + lax.axis_index("c")
        pltpu.sync_copy(pe_hbm, pe_v)
        pltpu.sync_copy(x_hbm.at[wid], idx_all)

        def issue_gather(u, p):
            pltpu.async_copy(table_hbm.at[idx_all.at[u]], rows[p], gsems[p])

        def wait_gather(u, p):
            pltpu.make_async_copy(
                table_hbm.at[idx_all.at[u]], rows[p], gsems[p]).wait()

        def issue_out(u, p):
            pltpu.async_copy(rows[p], out_hbm.at[wid, u], osems[p])

        def wait_out(u, p):
            pltpu.make_async_copy(rows[p], out_hbm.at[wid, u], osems[p]).wait()

        def compute(p):
            rowsb = rows[p]
            off = (p % 2) * U_ROWS

            @plsc.parallel_loop(0, U_ROWS, step=2)
            def _(r):
                for rr in range(2):
                    for c in range(NVEC):
                        sl = pl.ds(c * LANES, LANES)
                        rowsb[r + rr, sl] = (
                            rowsb[r + rr, sl] * _SCALE + pe_v[off + r + rr, sl])

        def unit(u, p, issue_a, wait_o):
            a = u + AHEAD
            pa = (p + AHEAD) % NBUF
            if wait_o:
                wait_out(a - NBUF, pa)
            if issue_a:
                issue_gather(a, pa)
            wait_gather(u, p)
            compute(p)
            issue_out(u, p)

        for u in range(AHEAD):
            issue_gather(u, u % NBUF)
        for p in range(NBUF):
            unit(p, p, True, p + AHEAD >= NBUF)

        def macro(j, _):
            u0 = NBUF * j
            for p in range(NBUF):
                unit(u0 + p, p, True, True)
            return ()

        lax.fori_loop(1, UNITS // NBUF - 1, macro, ())
        u0 = UNITS - NBUF
        for p in range(NBUF):
            unit(u0 + p, p, u0 + p + AHEAD < UNITS, u0 + p + AHEAD < UNITS)
        for u in range(UNITS - NBUF, UNITS):
            wait_out(u, u % NBUF)

    return k(x4, pe, table)


def kernel(x, table):
    x4 = x.reshape(NW, UNITS, U_ROWS)
    pe = _pos_encoding()
    out = _embed_encode(x4, pe, table)
    return out.reshape(BATCH, MAX_SEQ_LEN, D_MODEL)


# speedup vs baseline: 1.1138x; 1.1138x over previous
import functools
import math

import jax
import jax.numpy as jnp
from jax import lax
from jax.experimental import pallas as pl
from jax.experimental.pallas import tpu as pltpu
from jax.experimental.pallas import tpu_sc as plsc

D_MODEL = 128
MAX_SEQ_LEN = 200
BATCH = 1024
_SCALE = math.sqrt(float(D_MODEL))

NC = 2
NS = 16
NW = NC * NS
LANES = 16
NVEC = D_MODEL // LANES
U_ROWS = 100
UNITS = BATCH * MAX_SEQ_LEN // (NW * U_ROWS)
NBUF = 4
AHEAD = 2


def _pos_encoding():
    even_i = jnp.arange(0, D_MODEL, 2, dtype=jnp.float32)
    denominator = jnp.power(even_i, even_i / D_MODEL)
    position = jnp.arange(MAX_SEQ_LEN, dtype=jnp.float32).reshape(MAX_SEQ_LEN, 1)
    even_pe = jnp.sin(position / denominator)
    odd_pe = jnp.cos(position / denominator)
    stacked = jnp.stack([even_pe, odd_pe], axis=-1)
    return stacked.reshape(MAX_SEQ_LEN, D_MODEL)


def _embed_encode(x4, pe, table):
    mesh = plsc.VectorSubcoreMesh(core_axis_name="c", subcore_axis_name="s")

    scratch = [
        pltpu.VMEM((MAX_SEQ_LEN, D_MODEL), jnp.float32),
        pltpu.VMEM((UNITS, U_ROWS), jnp.int32),
    ]
    scratch += [pltpu.VMEM((U_ROWS, D_MODEL), jnp.float32)
                for _ in range(NBUF)]
    scratch += [pltpu.SemaphoreType.DMA for _ in range(2 * NBUF)]

    @functools.partial(
        pl.kernel,
        out_type=jax.ShapeDtypeStruct((NW, UNITS, U_ROWS, D_MODEL), jnp.float32),
        mesh=mesh,
        scratch_types=scratch,
    )
    def k(x_hbm, pe_hbm, table_hbm, out_hbm, pe_v, idx_all, *rest):
        rows = rest[:NBUF]
        gsems = rest[NBUF:2 * NBUF]
        osems = rest[2 * NBUF:3 * NBUF]

        wid = lax.axis_index("s") * NC + lax.axis_index("c")
        pltpu.sync_copy(pe_hbm, pe_v)
        pltpu.sync_copy(x_hbm.at[wid], idx_all)

        def issue_gather(u, p):
            pltpu.async_copy(table_hbm.at[idx_all.at[u]], rows[p], gsems[p])

        def wait_gather(u, p):
            pltpu.make_async_copy(
                table_hbm.at[idx_all.at[u]], rows[p], gsems[p]).wait()

        def issue_out(u, p):
            pltpu.async_copy(rows[p], out_hbm.at[wid, u], osems[p])

        def wait_out(u, p):
            pltpu.make_async_copy(rows[p], out_hbm.at[wid, u], osems[p]).wait()

        def compute(p):
            rowsb = rows[p]
            off = (p % 2) * U_ROWS

            @plsc.parallel_loop(0, U_ROWS, step=2)
            def _(r):
                for rr in range(2):
                    for c in range(NVEC):
                        sl = pl.ds(c * LANES, LANES)
                        rowsb[r + rr, sl] = (
                            rowsb[r + rr, sl] * _SCALE + pe_v[off + r + rr, sl])

        def unit(u, p, issue_a, wait_o):
            a = u + AHEAD
            pa = (p + AHEAD) % NBUF
            if wait_o:
                wait_out(a - NBUF, pa)
            if issue_a:
                issue_gather(a, pa)
            wait_gather(u, p)
            compute(p)
            issue_out(u, p)

        for u in range(AHEAD):
            issue_gather(u, u % NBUF)
        for p in range(NBUF):
            unit(p, p, True, p + AHEAD >= NBUF)

        def macro(j, _):
            u0 = NBUF * j
            for p in range(NBUF):
                unit(u0 + p, p, True, True)
            return ()

        lax.fori_loop(1, UNITS // NBUF - 1, macro, ())
        u0 = UNITS - NBUF
        for p in range(NBUF):
            unit(u0 + p, p, u0 + p + AHEAD < UNITS, u0 + p + AHEAD < UNITS)
        for u in range(UNITS - NBUF, UNITS):
            wait_out(u, u % NBUF)

    return k(x4, pe, table)


def kernel(x, table):
    x4 = x.reshape(NW, UNITS, U_ROWS)
    pe = _pos_encoding()
    out = _embed_encode(x4, pe, table)
    return out.reshape(BATCH, MAX_SEQ_LEN, D_MODEL)


# speedup vs baseline: 1.9891x; 1.7859x over previous
import functools
import math

import jax
import jax.numpy as jnp
from jax import lax
from jax.experimental import pallas as pl
from jax.experimental.pallas import tpu as pltpu
from jax.experimental.pallas import tpu_sc as plsc

D_MODEL = 128
MAX_SEQ_LEN = 200
BATCH = 1024
_SCALE = math.sqrt(float(D_MODEL))

NC = 2
NS = 16
NW = NC * NS
ROWS_PER_W = BATCH // NW
HALF = MAX_SEQ_LEN // 2
LANES = 16
NVEC = D_MODEL // LANES
NBUF = 3


def _pos_encoding():
    even_i = jnp.arange(0, D_MODEL, 2, dtype=jnp.float32)
    denominator = jnp.power(even_i, even_i / D_MODEL)
    position = jnp.arange(MAX_SEQ_LEN, dtype=jnp.float32).reshape(MAX_SEQ_LEN, 1)
    even_pe = jnp.sin(position / denominator)
    odd_pe = jnp.cos(position / denominator)
    stacked = jnp.stack([even_pe, odd_pe], axis=-1)
    return stacked.reshape(MAX_SEQ_LEN, D_MODEL)


def _embed_encode(x4, pe, table):
    mesh = plsc.VectorSubcoreMesh(core_axis_name="c", subcore_axis_name="s")

    scratch = [
        pltpu.VMEM((MAX_SEQ_LEN, D_MODEL), jnp.float32),
        pltpu.VMEM((ROWS_PER_W, 2, HALF), jnp.int32),
    ]
    scratch += [pltpu.VMEM((MAX_SEQ_LEN, D_MODEL), jnp.float32)
                for _ in range(NBUF)]
    scratch += [pltpu.SemaphoreType.DMA for _ in range(2 * NBUF)]

    @functools.partial(
        pl.kernel,
        out_type=jax.ShapeDtypeStruct((BATCH, MAX_SEQ_LEN, D_MODEL), jnp.float32),
        mesh=mesh,
        scratch_types=scratch,
    )
    def k(x_hbm, pe_hbm, table_hbm, out_hbm, pe_v, idx_all, *rest):
        rows = rest[:NBUF]
        gsems = rest[NBUF:2 * NBUF]
        osems = rest[2 * NBUF:3 * NBUF]

        wid = lax.axis_index("s") * NC + lax.axis_index("c")
        base = wid * ROWS_PER_W
        pltpu.sync_copy(pe_hbm, pe_v)
        pltpu.sync_copy(x_hbm.at[wid], idx_all)

        def issue_gather(i, p):
            pltpu.async_copy(table_hbm.at[idx_all.at[i, 0]],
                             rows[p].at[pl.ds(0, HALF)], gsems[p])
            pltpu.async_copy(table_hbm.at[idx_all.at[i, 1]],
                             rows[p].at[pl.ds(HALF, HALF)], gsems[p])

        def wait_gather(i, p):
            pltpu.make_async_copy(table_hbm.at[idx_all.at[i, 0]],
                                  rows[p].at[pl.ds(0, HALF)], gsems[p]).wait()
            pltpu.make_async_copy(table_hbm.at[idx_all.at[i, 1]],
                                  rows[p].at[pl.ds(HALF, HALF)], gsems[p]).wait()

        def issue_out(i, p):
            pltpu.async_copy(rows[p], out_hbm.at[base + i], osems[p])

        def wait_out(i, p):
            pltpu.make_async_copy(rows[p], out_hbm.at[base + i], osems[p]).wait()

        def compute(p):
            rowsb = rows[p]

            @plsc.parallel_loop(0, MAX_SEQ_LEN, step=2)
            def _(r):
                for rr in range(2):
                    for c in range(NVEC):
                        sl = pl.ds(c * LANES, LANES)
                        rowsb[r + rr, sl] = (
                            rowsb[r + rr, sl] * _SCALE + pe_v[r + rr, sl])

        issue_gather(0, 0)
        for i in range(ROWS_PER_W):
            p = i % NBUF
            nxt = i + 1
            if nxt < ROWS_PER_W:
                pn = nxt % NBUF
                if nxt >= NBUF:
                    wait_out(nxt - NBUF, pn)
                issue_gather(nxt, pn)
            wait_gather(i, p)
            compute(p)
            issue_out(i, p)
        for i in range(ROWS_PER_W - NBUF, ROWS_PER_W):
            wait_out(i, i % NBUF)

    return k(x4, pe, table)


def kernel(x, table):
    x4 = x.reshape(NW, ROWS_PER_W, 2, HALF)
    pe = _pos_encoding()
    return _embed_encode(x4, pe, table)
